# baseline (speedup 1.0000x reference)
import functools

import jax
import jax.numpy as jnp
from jax import lax
from jax.experimental import pallas as pl
from jax.experimental.pallas import tpu as pltpu
from jax.experimental.pallas import tpu_sc as plsc

NC = 2
NS = 16
NW = NC * NS


@functools.lru_cache(maxsize=None)
def _make_gather(n_rows, v, d):
    mesh = plsc.VectorSubcoreMesh(core_axis_name="c", subcore_axis_name="s")

    @functools.partial(
        pl.kernel,
        out_type=jax.ShapeDtypeStruct((NW * n_rows, d), jnp.float32),
        mesh=mesh,
        scratch_types=[
            pltpu.VMEM((n_rows,), jnp.int32),
            pltpu.SemaphoreType.DMA,
        ],
    )
    def gather_kernel(idx_hbm, table_hbm, out_hbm, idx_v, gsem):
        wid = lax.axis_index("s") * NC + lax.axis_index("c")
        base = wid * n_rows
        pltpu.sync_copy(idx_hbm.at[pl.ds(base, n_rows)], idx_v)

        @pl.loop(0, n_rows // 16)
        def _(g):
            iv = idx_v[pl.ds(g * 16, 16)]
            for i in range(16):
                pltpu.async_copy(
                    table_hbm.at[pl.ds(iv[i], 1)],
                    out_hbm.at[pl.ds(base + g * 16 + i, 1)],
                    gsem,
                )

        pltpu.make_async_copy(
            table_hbm.at[pl.ds(0, n_rows)],
            out_hbm.at[pl.ds(base, n_rows)],
            gsem,
        ).wait()

    return gather_kernel


def kernel(sequence, table):
    b, s = sequence.shape
    v, d = table.shape
    flat = sequence.reshape(-1).astype(jnp.int32)
    n = flat.shape[0]
    assert n % (NW * 16) == 0
    out = _make_gather(n // NW, v, d)(flat, table)
    return out.reshape(b, s, d)


# speedup vs baseline: 7.4384x; 7.4384x over previous
import functools

import jax
import jax.numpy as jnp
from jax import lax
from jax.experimental import pallas as pl
from jax.experimental.pallas import tpu as pltpu
from jax.experimental.pallas import tpu_sc as plsc

NC = 2
NS = 16
NW = NC * NS
CHUNK = 320
NBUF = 3


@functools.lru_cache(maxsize=None)
def _make_gather(n_chunks, v, d):
    mesh = plsc.VectorSubcoreMesh(core_axis_name="c", subcore_axis_name="s")
    n_rows = n_chunks * CHUNK

    @functools.partial(
        pl.kernel,
        out_type=jax.ShapeDtypeStruct((NW * n_rows, d), jnp.float32),
        mesh=mesh,
        scratch_types=[
            pltpu.VMEM((n_rows,), jnp.int32),
            pltpu.VMEM((NBUF, CHUNK, d), jnp.float32),
            pltpu.SemaphoreType.DMA((NBUF,)),
            pltpu.SemaphoreType.DMA((NBUF,)),
        ],
        compiler_params=pltpu.CompilerParams(
            disable_bounds_checks=True, disable_semaphore_checks=True),
    )
    def gather_kernel(idx_hbm, table_hbm, out_hbm, idx_v, rows_v,
                      gsem, osem):
        wid = lax.axis_index("s") * NC + lax.axis_index("c")
        base = wid * n_rows

        pltpu.sync_copy(idx_hbm.at[pl.ds(base, n_rows)], idx_v)

        def stage_and_issue(c, b):
            @pl.loop(0, CHUNK // 16)
            def _(g):
                iv = idx_v[pl.ds(c * CHUNK + g * 16, 16)]
                for i in range(16):
                    pltpu.async_copy(
                        table_hbm.at[pl.ds(iv[i], 1)],
                        rows_v.at[b].at[pl.ds(g * 16 + i, 1)],
                        gsem.at[b],
                    )

        def drain(b):
            pltpu.make_async_copy(
                table_hbm.at[pl.ds(0, CHUNK)], rows_v.at[b], gsem.at[b]
            ).wait()

        outs = [None] * n_chunks
        for c in range(min(NBUF, n_chunks)):
            stage_and_issue(c, c)
        for c in range(n_chunks):
            b = c % NBUF
            drain(b)
            outs[c] = pltpu.async_copy(
                rows_v.at[b], out_hbm.at[pl.ds(base + c * CHUNK, CHUNK)],
                osem.at[b])
            if c + NBUF < n_chunks:
                outs[c].wait()
                stage_and_issue(c + NBUF, b)
        for c in range(max(0, n_chunks - NBUF), n_chunks):
            outs[c].wait()

    return gather_kernel


def kernel(sequence, table):
    b, s = sequence.shape
    v, d = table.shape
    flat = sequence.reshape(-1).astype(jnp.int32)
    n = flat.shape[0]
    per_w = -(-n // (NW * CHUNK)) * CHUNK
    n_pad = NW * per_w
    if n_pad != n:
        flat = jnp.pad(flat, (0, n_pad - n))
    out = _make_gather(per_w // CHUNK, v, d)(flat, table)
    return out[:n].reshape(b, s, d)
